# Initial kernel scaffold; baseline (speedup 1.0000x reference)
import jax
import jax.numpy as jnp
from jax.experimental import pallas as pl


def kernel(inputs, embedding):
    raise NotImplementedError("write your pallas kernel here")


# speedup vs baseline: 1.0464x; 1.0464x over previous
import functools

import jax
import jax.numpy as jnp
from jax.experimental import pallas as pl
from jax.experimental.pallas import tpu as pltpu


def _vq_block(x_ref, emb_ref, q_ref, idx_ref, loss_ref):
    z = x_ref[0]
    emb = emb_ref[...]
    ne = emb.shape[0]

    scores = jax.lax.dot_general(
        z.astype(jnp.bfloat16), emb.astype(jnp.bfloat16),
        (((0,), (1,)), ((), ())),
        preferred_element_type=jnp.float32)
    e_norm = jnp.sum(emb * emb, axis=1)
    x_norm = jnp.sum(z * z, axis=0)
    dist = (x_norm[:, None] + e_norm[None, :]) - 2.0 * scores

    idx = jnp.argmin(dist, axis=1).astype(jnp.int32)

    cols = jax.lax.broadcasted_iota(jnp.int32, dist.shape, 1)
    onehot = (cols == idx[:, None]).astype(jnp.bfloat16)
    q = jax.lax.dot_general(
        emb.astype(jnp.bfloat16), onehot, (((0,), (1,)), ((), ())),
        preferred_element_type=jnp.float32)

    q_ref[0] = q
    idx_ref[0, 0] = idx

    @pl.when(pl.program_id(0) == 0)
    def _():
        loss_ref[...] = jnp.zeros_like(loss_ref)
    diff = q - z
    part = jnp.sum(diff * diff, axis=1, keepdims=True)
    loss_ref[...] += jnp.sum(part, axis=0, keepdims=True)


def kernel(inputs, embedding):
    b, d, t = inputs.shape
    ne = embedding.shape[0]
    tb = 256
    tpb = t // tb
    grid = b * tpb
    commitment_cost = 0.5

    q, idx, loss_sum = pl.pallas_call(
        _vq_block,
        grid=(grid,),
        in_specs=[
            pl.BlockSpec((1, d, tb), lambda g: (g // tpb, 0, g % tpb)),
            pl.BlockSpec((ne, d), lambda g: (0, 0)),
        ],
        out_specs=[
            pl.BlockSpec((1, d, tb), lambda g: (g // tpb, 0, g % tpb)),
            pl.BlockSpec((1, 1, tb), lambda g: (g, 0, 0)),
            pl.BlockSpec((1, 1), lambda g: (0, 0)),
        ],
        out_shape=[
            jax.ShapeDtypeStruct((b, d, t), jnp.float32),
            jax.ShapeDtypeStruct((grid, 1, tb), jnp.int32),
            jax.ShapeDtypeStruct((1, 1), jnp.float32),
        ],
    )(inputs, embedding)

    loss = commitment_cost * loss_sum[0, 0] / (b * t * d)
    return q, loss, idx.reshape(b * t, 1)


# speedup vs baseline: 1.4512x; 1.3869x over previous
import functools

import jax
import jax.numpy as jnp
from jax.experimental import pallas as pl
from jax.experimental.pallas import tpu as pltpu


def _vq_block(x_ref, emb_ref, q_ref, idx_ref, loss_ref):
    z = x_ref[0]
    emb = emb_ref[...]
    ne = emb.shape[0]

    scores = jax.lax.dot_general(
        z.astype(jnp.bfloat16), emb.astype(jnp.bfloat16),
        (((0,), (1,)), ((), ())),
        preferred_element_type=jnp.float32)
    e_norm = jnp.sum(emb * emb, axis=1)
    x_norm = jnp.sum(z * z, axis=0)
    dist = (x_norm[:, None] + e_norm[None, :]) - 2.0 * scores

    idx = jnp.argmin(dist, axis=1).astype(jnp.int32)

    cols = jax.lax.broadcasted_iota(jnp.int32, dist.shape, 1)
    onehot = (cols == idx[:, None]).astype(jnp.bfloat16)
    q = jax.lax.dot_general(
        emb.astype(jnp.bfloat16), onehot, (((0,), (1,)), ((), ())),
        preferred_element_type=jnp.float32)

    q_ref[0] = q
    idx_ref[0, 0] = idx

    @pl.when(pl.program_id(0) == 0)
    def _():
        loss_ref[...] = jnp.zeros_like(loss_ref)
    diff = q - z
    part = jnp.sum(diff * diff, axis=1, keepdims=True)
    loss_ref[...] += jnp.sum(part, axis=0, keepdims=True)


def kernel(inputs, embedding):
    b, d, t = inputs.shape
    ne = embedding.shape[0]
    tb = 512
    tpb = t // tb
    grid = b * tpb
    commitment_cost = 0.5

    q, idx, loss_sum = pl.pallas_call(
        _vq_block,
        grid=(grid,),
        in_specs=[
            pl.BlockSpec((1, d, tb), lambda g: (g // tpb, 0, g % tpb)),
            pl.BlockSpec((ne, d), lambda g: (0, 0)),
        ],
        out_specs=[
            pl.BlockSpec((1, d, tb), lambda g: (g // tpb, 0, g % tpb)),
            pl.BlockSpec((1, 1, tb), lambda g: (g, 0, 0)),
            pl.BlockSpec((1, 1), lambda g: (0, 0)),
        ],
        out_shape=[
            jax.ShapeDtypeStruct((b, d, t), jnp.float32),
            jax.ShapeDtypeStruct((grid, 1, tb), jnp.int32),
            jax.ShapeDtypeStruct((1, 1), jnp.float32),
        ],
    )(inputs, embedding)

    loss = commitment_cost * loss_sum[0, 0] / (b * t * d)
    return q, loss, idx.reshape(b * t, 1)
